# Initial kernel scaffold; baseline (speedup 1.0000x reference)
import jax
import jax.numpy as jnp
from jax.experimental import pallas as pl


def kernel(out, z, batch):
    raise NotImplementedError("write your pallas kernel here")


# speedup vs baseline: 154.8296x; 154.8296x over previous
import functools

import jax
import jax.numpy as jnp
from jax import lax
from jax.experimental import pallas as pl
from jax.experimental.pallas import tpu as pltpu
from jax.experimental.pallas import tpu_sc as plsc

N = 10000
D = 128
NUM_GRAPHS = 64
NUM_ELEMENTS = 100
SEGS = NUM_GRAPHS * NUM_ELEMENTS
HALF = SEGS // 2
DUMMY = HALF
CH = 80
NCH = N // CH
NTILES = 16
CHUNKS_PER_TILE = (NCH + NTILES - 1) // NTILES
ZROWS = 40
OUT_ROWS_PER_TILE = HALF // NTILES


def _pool_kernel(out_hbm, z_hbm, b_hbm, o_hbm, zerobuf, zc, bc, idx, feat, acc):
    cid = lax.axis_index("c")
    sid = lax.axis_index("s")

    def zero_body(k, _):
        zerobuf[k, pl.ds(0, 16)] = jnp.zeros((16,), jnp.float32)
        zerobuf[k, pl.ds(16, 16)] = jnp.zeros((16,), jnp.float32)
        zerobuf[k, pl.ds(32, 16)] = jnp.zeros((16,), jnp.float32)
        zerobuf[k, pl.ds(48, 16)] = jnp.zeros((16,), jnp.float32)
        zerobuf[k, pl.ds(64, 16)] = jnp.zeros((16,), jnp.float32)
        zerobuf[k, pl.ds(80, 16)] = jnp.zeros((16,), jnp.float32)
        zerobuf[k, pl.ds(96, 16)] = jnp.zeros((16,), jnp.float32)
        zerobuf[k, pl.ds(112, 16)] = jnp.zeros((16,), jnp.float32)
        return 0

    lax.fori_loop(0, ZROWS, zero_body, 0)

    def zcopy_body(i, _):
        pltpu.sync_copy(
            zerobuf,
            acc.at[pl.ds(sid * OUT_ROWS_PER_TILE + i * ZROWS, ZROWS), :],
        )
        return 0

    lax.fori_loop(0, OUT_ROWS_PER_TILE // ZROWS, zcopy_body, 0)
    plsc.subcore_barrier()

    half_base = cid * HALF

    def chunk_body(j, _):
        c = j * NTILES + sid

        @pl.when(c < NCH)
        def _():
            base = c * CH
            pltpu.sync_copy(z_hbm.at[pl.ds(base, CH)], zc)
            pltpu.sync_copy(b_hbm.at[pl.ds(base, CH)], bc)

            def idx_body(k, _):
                zv = zc[pl.ds(k * 16, 16)]
                bv = bc[pl.ds(k * 16, 16)]
                seg = bv * NUM_ELEMENTS + zv - 1
                loc = seg - half_base
                ok = (loc >= 0) & (loc < HALF)
                idx[pl.ds(k * 16, 16)] = jnp.where(ok, loc, DUMMY)
                return 0

            lax.fori_loop(0, CH // 16, idx_body, 0)
            pltpu.sync_copy(out_hbm.at[pl.ds(base, CH), :], feat)
            pltpu.sync_copy(feat, acc.at[idx], add=True)

        return 0

    lax.fori_loop(0, CHUNKS_PER_TILE, chunk_body, 0)
    plsc.subcore_barrier()

    pltpu.sync_copy(
        acc.at[pl.ds(sid * OUT_ROWS_PER_TILE, OUT_ROWS_PER_TILE), :],
        o_hbm.at[pl.ds(cid * HALF + sid * OUT_ROWS_PER_TILE, OUT_ROWS_PER_TILE), :],
    )


@jax.jit
def _pool(out, z, batch):
    mesh = plsc.VectorSubcoreMesh(core_axis_name="c", subcore_axis_name="s")
    f = functools.partial(
        pl.kernel,
        out_type=jax.ShapeDtypeStruct((SEGS, D), jnp.float32),
        mesh=mesh,
        scratch_types=[
            pltpu.VMEM((ZROWS, D), jnp.float32),
            pltpu.VMEM((CH,), jnp.int32),
            pltpu.VMEM((CH,), jnp.int32),
            pltpu.VMEM((CH,), jnp.int32),
            pltpu.VMEM((CH, D), jnp.float32),
            pltpu.VMEM_SHARED((HALF + 1, D), jnp.float32),
        ],
    )(_pool_kernel)
    return f(out, z, batch)


def kernel(out, z, batch):
    pooled = _pool(out, z.astype(jnp.int32), batch.astype(jnp.int32))
    return pooled.reshape(NUM_GRAPHS, NUM_ELEMENTS * D)


# speedup vs baseline: 218.5913x; 1.4118x over previous
import functools

import jax
import jax.numpy as jnp
from jax import lax
from jax.experimental import pallas as pl
from jax.experimental.pallas import tpu as pltpu
from jax.experimental.pallas import tpu_sc as plsc

N = 10000
D = 128
NUM_GRAPHS = 64
NUM_ELEMENTS = 100
SEGS = NUM_GRAPHS * NUM_ELEMENTS
CH = 80
NCH = N // CH
NTILES = 16
CPT = (NCH + NTILES - 1) // NTILES
RING = 4
ZROWS = 40
ACC_PER_TILE = SEGS // NTILES


def _pool_kernel(out_hbm, z_hbm, b_hbm, o_hbm,
                 zerobuf, zc, bc, idx2, feat, acc, sem_zb, sem_f):
    sid = lax.axis_index("s")

    for j in range(RING):
        c = j * NTILES + sid

        @pl.when(c < NCH)
        def _(j=j, c=c):
            pltpu.async_copy(out_hbm.at[pl.ds(c * CH, CH), :], feat.at[j],
                             sem_f.at[j])

    for j in range(CPT):
        c = j * NTILES + sid

        @pl.when(c < NCH)
        def _(j=j, c=c):
            pltpu.async_copy(z_hbm.at[pl.ds(c * CH, CH)],
                             zc.at[pl.ds(j * CH, CH)], sem_zb)
            pltpu.async_copy(b_hbm.at[pl.ds(c * CH, CH)],
                             bc.at[pl.ds(j * CH, CH)], sem_zb)

    def zero_body(k, _):
        for q in range(D // 16):
            zerobuf[k, pl.ds(q * 16, 16)] = jnp.zeros((16,), jnp.float32)
        return 0

    lax.fori_loop(0, ZROWS, zero_body, 0)

    def zcopy_body(i, _):
        pltpu.sync_copy(zerobuf,
                        acc.at[pl.ds(sid * ACC_PER_TILE + i * ZROWS, ZROWS), :])
        return 0

    lax.fori_loop(0, ACC_PER_TILE // ZROWS, zcopy_body, 0)

    for j in range(CPT):
        c = j * NTILES + sid

        @pl.when(c < NCH)
        def _(j=j, c=c):
            pltpu.make_async_copy(z_hbm.at[pl.ds(c * CH, CH)],
                                  zc.at[pl.ds(j * CH, CH)], sem_zb).wait()
            pltpu.make_async_copy(b_hbm.at[pl.ds(c * CH, CH)],
                                  bc.at[pl.ds(j * CH, CH)], sem_zb).wait()

            def idx_body(k, _):
                zv = zc[pl.ds(j * CH + k * 16, 16)]
                bv = bc[pl.ds(j * CH + k * 16, 16)]
                idx2[j, pl.ds(k * 16, 16)] = bv * NUM_ELEMENTS + zv - 1
                return 0

            lax.fori_loop(0, CH // 16, idx_body, 0)

    plsc.subcore_barrier()

    for j in range(CPT):
        c = j * NTILES + sid
        s = j % RING

        @pl.when(c < NCH)
        def _(j=j, c=c, s=s):
            pltpu.make_async_copy(out_hbm.at[pl.ds(c * CH, CH), :], feat.at[s],
                                  sem_f.at[s]).wait()
            pltpu.sync_copy(feat.at[s], acc.at[idx2.at[j]], add=True)

        if j + RING < CPT:
            c2 = (j + RING) * NTILES + sid

            @pl.when(c2 < NCH)
            def _(j=j, c2=c2, s=s):
                pltpu.async_copy(out_hbm.at[pl.ds(c2 * CH, CH), :], feat.at[s],
                                 sem_f.at[s])

    plsc.subcore_barrier()

    pltpu.sync_copy(acc.at[pl.ds(sid * ACC_PER_TILE, ACC_PER_TILE), :],
                    o_hbm.at[pl.ds(sid * ACC_PER_TILE, ACC_PER_TILE), :])


@jax.jit
def _pool(out, z, batch):
    mesh = plsc.VectorSubcoreMesh(core_axis_name="c", subcore_axis_name="s",
                                  num_cores=1)
    f = functools.partial(
        pl.kernel,
        out_type=jax.ShapeDtypeStruct((SEGS, D), jnp.float32),
        mesh=mesh,
        scratch_types=[
            pltpu.VMEM((ZROWS, D), jnp.float32),
            pltpu.VMEM((CPT * CH,), jnp.int32),
            pltpu.VMEM((CPT * CH,), jnp.int32),
            pltpu.VMEM((CPT, CH), jnp.int32),
            pltpu.VMEM((RING, CH, D), jnp.float32),
            pltpu.VMEM_SHARED((SEGS, D), jnp.float32),
            pltpu.SemaphoreType.DMA,
            pltpu.SemaphoreType.DMA((RING,)),
        ],
    )(_pool_kernel)
    return f(out, z, batch)


def kernel(out, z, batch):
    pooled = _pool(out, z.astype(jnp.int32), batch.astype(jnp.int32))
    return pooled.reshape(NUM_GRAPHS, NUM_ELEMENTS * D)


# speedup vs baseline: 224.7660x; 1.0282x over previous
import functools

import jax
import jax.numpy as jnp
from jax import lax
from jax.experimental import pallas as pl
from jax.experimental.pallas import tpu as pltpu
from jax.experimental.pallas import tpu_sc as plsc

N = 10000
D = 128
NUM_GRAPHS = 64
NUM_ELEMENTS = 100
SEGS = NUM_GRAPHS * NUM_ELEMENTS
HALF = SEGS // 2
GHALF = NUM_GRAPHS // 2
DUMMY = HALF
CH = 80
NCH = N // CH
NTILES = 16
CPT = (NCH + NTILES - 1) // NTILES
RING = 4
ZROWS = 40
ACC_PER_TILE = HALF // NTILES


def _pool_kernel(out_hbm, z_hbm, b_hbm, o_hbm,
                 zerobuf, zc, bc, idx2, feat, acc, sem_zb, sem_f):
    cid = lax.axis_index("c")
    sid = lax.axis_index("s")
    g_lo = cid * GHALF

    for j in range(CPT):
        c = j * NTILES + sid

        @pl.when(c < NCH)
        def _(j=j, c=c):
            pltpu.async_copy(z_hbm.at[pl.ds(c * CH, CH)],
                             zc.at[pl.ds(j * CH, CH)], sem_zb)
            pltpu.async_copy(b_hbm.at[pl.ds(c * CH, CH)],
                             bc.at[pl.ds(j * CH, CH)], sem_zb)

    def zero_body(k, _):
        for q in range(D // 16):
            zerobuf[k, pl.ds(q * 16, 16)] = jnp.zeros((16,), jnp.float32)
        return 0

    lax.fori_loop(0, ZROWS, zero_body, 0)

    def zcopy_body(i, _):
        pltpu.sync_copy(zerobuf,
                        acc.at[pl.ds(sid * ACC_PER_TILE + i * ZROWS, ZROWS), :])
        return 0

    lax.fori_loop(0, ACC_PER_TILE // ZROWS, zcopy_body, 0)

    owned = []
    for j in range(CPT):
        c = j * NTILES + sid
        in_range = c < NCH

        @pl.when(in_range)
        def _(j=j, c=c):
            pltpu.make_async_copy(z_hbm.at[pl.ds(c * CH, CH)],
                                  zc.at[pl.ds(j * CH, CH)], sem_zb).wait()
            pltpu.make_async_copy(b_hbm.at[pl.ds(c * CH, CH)],
                                  bc.at[pl.ds(j * CH, CH)], sem_zb).wait()

            def idx_body(k, _):
                zv = zc[pl.ds(j * CH + k * 16, 16)]
                bv = bc[pl.ds(j * CH + k * 16, 16)]
                loc = (bv - g_lo) * NUM_ELEMENTS + zv - 1
                ok = (loc >= 0) & (loc < HALF)
                idx2[j, pl.ds(k * 16, 16)] = jnp.where(ok, loc, DUMMY)
                return 0

            lax.fori_loop(0, CH // 16, idx_body, 0)

        b_first = bc[pl.ds(j * CH, 16)][0]
        b_last = bc[pl.ds(j * CH + CH - 16, 16)][15]
        owned.append(jnp.logical_and(
            in_range,
            jnp.logical_and(b_last >= g_lo, b_first < g_lo + GHALF)))

    for j in range(RING):
        c = j * NTILES + sid

        @pl.when(owned[j])
        def _(j=j, c=c):
            pltpu.async_copy(out_hbm.at[pl.ds(c * CH, CH), :], feat.at[j],
                             sem_f.at[j])

    plsc.subcore_barrier()

    for j in range(CPT):
        c = j * NTILES + sid
        s = j % RING

        @pl.when(owned[j])
        def _(j=j, c=c, s=s):
            pltpu.make_async_copy(out_hbm.at[pl.ds(c * CH, CH), :], feat.at[s],
                                  sem_f.at[s]).wait()
            pltpu.sync_copy(feat.at[s], acc.at[idx2.at[j]], add=True)

        if j + RING < CPT:
            c2 = (j + RING) * NTILES + sid

            @pl.when(owned[j + RING])
            def _(j=j, c2=c2, s=s):
                pltpu.async_copy(out_hbm.at[pl.ds(c2 * CH, CH), :], feat.at[s],
                                 sem_f.at[s])

    plsc.subcore_barrier()

    pltpu.sync_copy(
        acc.at[pl.ds(sid * ACC_PER_TILE, ACC_PER_TILE), :],
        o_hbm.at[pl.ds(cid * HALF + sid * ACC_PER_TILE, ACC_PER_TILE), :])


@jax.jit
def _pool(out, z, batch):
    mesh = plsc.VectorSubcoreMesh(core_axis_name="c", subcore_axis_name="s",
                                  num_cores=2)
    f = functools.partial(
        pl.kernel,
        out_type=jax.ShapeDtypeStruct((SEGS, D), jnp.float32),
        mesh=mesh,
        scratch_types=[
            pltpu.VMEM((ZROWS, D), jnp.float32),
            pltpu.VMEM((CPT * CH,), jnp.int32),
            pltpu.VMEM((CPT * CH,), jnp.int32),
            pltpu.VMEM((CPT, CH), jnp.int32),
            pltpu.VMEM((RING, CH, D), jnp.float32),
            pltpu.VMEM_SHARED((HALF + 1, D), jnp.float32),
            pltpu.SemaphoreType.DMA,
            pltpu.SemaphoreType.DMA((RING,)),
        ],
    )(_pool_kernel)
    return f(out, z, batch)


def kernel(out, z, batch):
    pooled = _pool(out, z.astype(jnp.int32), batch.astype(jnp.int32))
    return pooled.reshape(NUM_GRAPHS, NUM_ELEMENTS * D)


# speedup vs baseline: 229.2029x; 1.0197x over previous
import functools

import jax
import jax.numpy as jnp
from jax import lax
from jax.experimental import pallas as pl
from jax.experimental.pallas import tpu as pltpu
from jax.experimental.pallas import tpu_sc as plsc

N = 10000
D = 128
NUM_GRAPHS = 64
NUM_ELEMENTS = 100
SEGS = NUM_GRAPHS * NUM_ELEMENTS
HALF = SEGS // 2
GHALF = NUM_GRAPHS // 2
DUMMY = HALF
CH = 80
NCH = N // CH
NTILES = 16
CPT = (NCH + NTILES - 1) // NTILES
RING = 4
ZROWS = 40
ACC_PER_TILE = HALF // NTILES


def _pool_kernel(out_hbm, z_hbm, b_hbm, o_hbm,
                 zerobuf, zc, bc, idx2, feat, acc, sem_zb, sem_z0, sem_f,
                 sem_sc):
    cid = lax.axis_index("c")
    sid = lax.axis_index("s")
    g_lo = cid * GHALF

    for j in range(CPT):
        c = j * NTILES + sid

        @pl.when(c < NCH)
        def _(j=j, c=c):
            pltpu.async_copy(z_hbm.at[pl.ds(c * CH, CH)],
                             zc.at[pl.ds(j * CH, CH)], sem_zb)
            pltpu.async_copy(b_hbm.at[pl.ds(c * CH, CH)],
                             bc.at[pl.ds(j * CH, CH)], sem_zb)

    def zero_body(k, _):
        for q in range(D // 16):
            zerobuf[k, pl.ds(q * 16, 16)] = jnp.zeros((16,), jnp.float32)
        return 0

    lax.fori_loop(0, ZROWS, zero_body, 0)
    for i in range(ACC_PER_TILE // ZROWS):
        pltpu.async_copy(
            zerobuf, acc.at[pl.ds(sid * ACC_PER_TILE + i * ZROWS, ZROWS), :],
            sem_z0)

    owned = []
    for j in range(CPT):
        c = j * NTILES + sid
        in_range = c < NCH

        @pl.when(in_range)
        def _(j=j, c=c):
            pltpu.make_async_copy(z_hbm.at[pl.ds(c * CH, CH)],
                                  zc.at[pl.ds(j * CH, CH)], sem_zb).wait()
            pltpu.make_async_copy(b_hbm.at[pl.ds(c * CH, CH)],
                                  bc.at[pl.ds(j * CH, CH)], sem_zb).wait()

        b_first = bc[pl.ds(j * CH, 16)][0]
        b_last = bc[pl.ds(j * CH + CH - 16, 16)][15]
        own = jnp.logical_and(
            in_range,
            jnp.logical_and(b_last >= g_lo, b_first < g_lo + GHALF))
        owned.append(own)

        @pl.when(own)
        def _(j=j):
            def idx_body(k, _):
                zv = zc[pl.ds(j * CH + k * 16, 16)]
                bv = bc[pl.ds(j * CH + k * 16, 16)]
                loc = (bv - g_lo) * NUM_ELEMENTS + zv - 1
                ok = (loc >= 0) & (loc < HALF)
                idx2[j, pl.ds(k * 16, 16)] = jnp.where(ok, loc, DUMMY)
                return 0

            lax.fori_loop(0, CH // 16, idx_body, 0)

    for j in range(RING):
        c = j * NTILES + sid

        @pl.when(owned[j])
        def _(j=j, c=c):
            pltpu.async_copy(out_hbm.at[pl.ds(c * CH, CH), :], feat.at[j],
                             sem_f.at[j])

    for i in range(ACC_PER_TILE // ZROWS):
        pltpu.make_async_copy(
            zerobuf, acc.at[pl.ds(sid * ACC_PER_TILE + i * ZROWS, ZROWS), :],
            sem_z0).wait()
    plsc.subcore_barrier()

    def scatter_desc(j, s):
        return pltpu.make_async_copy(feat.at[s], acc.at[idx2.at[j]],
                                     sem_sc.at[s])

    for j in range(CPT):
        c = j * NTILES + sid
        s = j % RING

        @pl.when(owned[j])
        def _(j=j, c=c, s=s):
            pltpu.make_async_copy(out_hbm.at[pl.ds(c * CH, CH), :], feat.at[s],
                                  sem_f.at[s]).wait()
            pltpu.async_copy(feat.at[s], acc.at[idx2.at[j]], sem_sc.at[s],
                             add=True)

        if j + RING < CPT:
            c2 = (j + RING) * NTILES + sid

            @pl.when(owned[j + RING])
            def _(j=j, c2=c2, s=s):
                @pl.when(owned[j])
                def _():
                    scatter_desc(j, s).wait()

                pltpu.async_copy(out_hbm.at[pl.ds(c2 * CH, CH), :], feat.at[s],
                                 sem_f.at[s])

    for j in range(CPT):
        s = j % RING
        if j + RING < CPT:
            not_refilled = jnp.logical_not(owned[j + RING])
            pred = jnp.logical_and(owned[j], not_refilled)
        else:
            pred = owned[j]

        @pl.when(pred)
        def _(j=j, s=s):
            scatter_desc(j, s).wait()

    plsc.subcore_barrier()

    pltpu.sync_copy(
        acc.at[pl.ds(sid * ACC_PER_TILE, ACC_PER_TILE), :],
        o_hbm.at[pl.ds(cid * HALF + sid * ACC_PER_TILE, ACC_PER_TILE), :])


@jax.jit
def _pool(out, z, batch):
    mesh = plsc.VectorSubcoreMesh(core_axis_name="c", subcore_axis_name="s",
                                  num_cores=2)
    f = functools.partial(
        pl.kernel,
        out_type=jax.ShapeDtypeStruct((SEGS, D), jnp.float32),
        mesh=mesh,
        scratch_types=[
            pltpu.VMEM((ZROWS, D), jnp.float32),
            pltpu.VMEM((CPT * CH,), jnp.int32),
            pltpu.VMEM((CPT * CH,), jnp.int32),
            pltpu.VMEM((CPT, CH), jnp.int32),
            pltpu.VMEM((RING, CH, D), jnp.float32),
            pltpu.VMEM_SHARED((HALF + 1, D), jnp.float32),
            pltpu.SemaphoreType.DMA,
            pltpu.SemaphoreType.DMA,
            pltpu.SemaphoreType.DMA((RING,)),
            pltpu.SemaphoreType.DMA((RING,)),
        ],
    )(_pool_kernel)
    return f(out, z, batch)


def kernel(out, z, batch):
    pooled = _pool(out, z.astype(jnp.int32), batch.astype(jnp.int32))
    return pooled.reshape(NUM_GRAPHS, NUM_ELEMENTS * D)


# speedup vs baseline: 230.2765x; 1.0047x over previous
import functools

import jax
import jax.numpy as jnp
from jax import lax
from jax.experimental import pallas as pl
from jax.experimental.pallas import tpu as pltpu
from jax.experimental.pallas import tpu_sc as plsc

N = 10000
D = 128
NUM_GRAPHS = 64
NUM_ELEMENTS = 100
SEGS = NUM_GRAPHS * NUM_ELEMENTS
HALF = SEGS // 2
GHALF = NUM_GRAPHS // 2
DUMMY = HALF
CH = 80
NCH = N // CH
NTILES = 16
CPT = (NCH + NTILES - 1) // NTILES
RING = 8
ZROWS = 40
ACC_PER_TILE = HALF // NTILES


def _pool_kernel(out_hbm, z_hbm, b_hbm, o_hbm,
                 zerobuf, zc, bc, idx2, feat, acc, sem_zb, sem_z0, sem_f,
                 sem_sc):
    cid = lax.axis_index("c")
    sid = lax.axis_index("s")
    g_lo = cid * GHALF

    for j in range(CPT):
        c = j * NTILES + sid

        @pl.when(c < NCH)
        def _(j=j, c=c):
            pltpu.async_copy(z_hbm.at[pl.ds(c * CH, CH)],
                             zc.at[pl.ds(j * CH, CH)], sem_zb)
            pltpu.async_copy(b_hbm.at[pl.ds(c * CH, CH)],
                             bc.at[pl.ds(j * CH, CH)], sem_zb)

    def zero_body(k, _):
        for q in range(D // 16):
            zerobuf[k, pl.ds(q * 16, 16)] = jnp.zeros((16,), jnp.float32)
        return 0

    lax.fori_loop(0, ZROWS, zero_body, 0)
    for i in range(ACC_PER_TILE // ZROWS):
        pltpu.async_copy(
            zerobuf, acc.at[pl.ds(sid * ACC_PER_TILE + i * ZROWS, ZROWS), :],
            sem_z0)

    owned = []
    for j in range(CPT):
        c = j * NTILES + sid
        in_range = c < NCH

        @pl.when(in_range)
        def _(j=j, c=c):
            pltpu.make_async_copy(z_hbm.at[pl.ds(c * CH, CH)],
                                  zc.at[pl.ds(j * CH, CH)], sem_zb).wait()
            pltpu.make_async_copy(b_hbm.at[pl.ds(c * CH, CH)],
                                  bc.at[pl.ds(j * CH, CH)], sem_zb).wait()

        b_first = bc[pl.ds(j * CH, 16)][0]
        b_last = bc[pl.ds(j * CH + CH - 16, 16)][15]
        own = jnp.logical_and(
            in_range,
            jnp.logical_and(b_last >= g_lo, b_first < g_lo + GHALF))
        owned.append(own)

        @pl.when(own)
        def _(j=j):
            def idx_body(k, _):
                zv = zc[pl.ds(j * CH + k * 16, 16)]
                bv = bc[pl.ds(j * CH + k * 16, 16)]
                loc = (bv - g_lo) * NUM_ELEMENTS + zv - 1
                ok = (loc >= 0) & (loc < HALF)
                idx2[j, pl.ds(k * 16, 16)] = jnp.where(ok, loc, DUMMY)
                return 0

            lax.fori_loop(0, CH // 16, idx_body, 0)

    for j in range(RING):
        c = j * NTILES + sid

        @pl.when(owned[j])
        def _(j=j, c=c):
            pltpu.async_copy(out_hbm.at[pl.ds(c * CH, CH), :], feat.at[j],
                             sem_f.at[j])

    for i in range(ACC_PER_TILE // ZROWS):
        pltpu.make_async_copy(
            zerobuf, acc.at[pl.ds(sid * ACC_PER_TILE + i * ZROWS, ZROWS), :],
            sem_z0).wait()
    plsc.subcore_barrier()

    def scatter_desc(j, s):
        return pltpu.make_async_copy(feat.at[s], acc.at[idx2.at[j]],
                                     sem_sc.at[s])

    for j in range(CPT):
        c = j * NTILES + sid
        s = j % RING

        @pl.when(owned[j])
        def _(j=j, c=c, s=s):
            pltpu.make_async_copy(out_hbm.at[pl.ds(c * CH, CH), :], feat.at[s],
                                  sem_f.at[s]).wait()
            pltpu.async_copy(feat.at[s], acc.at[idx2.at[j]], sem_sc.at[s],
                             add=True)

        if j + RING < CPT:
            c2 = (j + RING) * NTILES + sid

            @pl.when(owned[j + RING])
            def _(j=j, c2=c2, s=s):
                @pl.when(owned[j])
                def _():
                    scatter_desc(j, s).wait()

                pltpu.async_copy(out_hbm.at[pl.ds(c2 * CH, CH), :], feat.at[s],
                                 sem_f.at[s])

    for j in range(CPT):
        s = j % RING
        if j + RING < CPT:
            not_refilled = jnp.logical_not(owned[j + RING])
            pred = jnp.logical_and(owned[j], not_refilled)
        else:
            pred = owned[j]

        @pl.when(pred)
        def _(j=j, s=s):
            scatter_desc(j, s).wait()

    plsc.subcore_barrier()

    pltpu.sync_copy(
        acc.at[pl.ds(sid * ACC_PER_TILE, ACC_PER_TILE), :],
        o_hbm.at[pl.ds(cid * HALF + sid * ACC_PER_TILE, ACC_PER_TILE), :])


@jax.jit
def _pool(out, z, batch):
    mesh = plsc.VectorSubcoreMesh(core_axis_name="c", subcore_axis_name="s",
                                  num_cores=2)
    f = functools.partial(
        pl.kernel,
        out_type=jax.ShapeDtypeStruct((SEGS, D), jnp.float32),
        mesh=mesh,
        scratch_types=[
            pltpu.VMEM((ZROWS, D), jnp.float32),
            pltpu.VMEM((CPT * CH,), jnp.int32),
            pltpu.VMEM((CPT * CH,), jnp.int32),
            pltpu.VMEM((CPT, CH), jnp.int32),
            pltpu.VMEM((RING, CH, D), jnp.float32),
            pltpu.VMEM_SHARED((HALF + 1, D), jnp.float32),
            pltpu.SemaphoreType.DMA,
            pltpu.SemaphoreType.DMA,
            pltpu.SemaphoreType.DMA((RING,)),
            pltpu.SemaphoreType.DMA((RING,)),
        ],
    )(_pool_kernel)
    return f(out, z, batch)


def kernel(out, z, batch):
    pooled = _pool(out, z.astype(jnp.int32), batch.astype(jnp.int32))
    return pooled.reshape(NUM_GRAPHS, NUM_ELEMENTS * D)


# speedup vs baseline: 230.7574x; 1.0021x over previous
import functools

import jax
import jax.numpy as jnp
from jax import lax
from jax.experimental import pallas as pl
from jax.experimental.pallas import tpu as pltpu
from jax.experimental.pallas import tpu_sc as plsc

N = 10000
D = 128
NUM_GRAPHS = 64
NUM_ELEMENTS = 100
SEGS = NUM_GRAPHS * NUM_ELEMENTS
HALF = SEGS // 2
GHALF = NUM_GRAPHS // 2
DUMMY = HALF
CH = 80
NCH = N // CH
NTILES = 16
CPT = (NCH + NTILES - 1) // NTILES
RING = 8
ZROWS = 40
ACC_PER_TILE = HALF // NTILES


def _pool_kernel(out_hbm, z_hbm, b_hbm, o_hbm,
                 zerobuf, zc, bc, idx2, feat, acc, sem_zb, sem_z0, sem_f,
                 sem_sc):
    cid = lax.axis_index("c")
    sid = lax.axis_index("s")
    g_lo = cid * GHALF

    for j in range(CPT):
        c = j * NTILES + sid

        @pl.when(c < NCH)
        def _(j=j, c=c):
            pltpu.async_copy(z_hbm.at[pl.ds(c * CH, CH)],
                             zc.at[pl.ds(j * CH, CH)], sem_zb)
            pltpu.async_copy(b_hbm.at[pl.ds(c * CH, CH)],
                             bc.at[pl.ds(j * CH, CH)], sem_zb)

    def zero_body(k, _):
        for q in range(D // 16):
            zerobuf[k, pl.ds(q * 16, 16)] = jnp.zeros((16,), jnp.float32)
        return 0

    lax.fori_loop(0, ZROWS, zero_body, 0)
    for i in range(ACC_PER_TILE // ZROWS):
        pltpu.async_copy(
            zerobuf, acc.at[pl.ds(sid * ACC_PER_TILE + i * ZROWS, ZROWS), :],
            sem_z0)

    owned = []
    for j in range(CPT):
        c = j * NTILES + sid
        in_range = c < NCH

        @pl.when(in_range)
        def _(j=j, c=c):
            pltpu.make_async_copy(z_hbm.at[pl.ds(c * CH, CH)],
                                  zc.at[pl.ds(j * CH, CH)], sem_zb).wait()
            pltpu.make_async_copy(b_hbm.at[pl.ds(c * CH, CH)],
                                  bc.at[pl.ds(j * CH, CH)], sem_zb).wait()

        b_first = bc[pl.ds(j * CH, 16)][0]
        b_last = bc[pl.ds(j * CH + CH - 16, 16)][15]
        own = jnp.logical_and(
            in_range,
            jnp.logical_and(b_last >= g_lo, b_first < g_lo + GHALF))
        owned.append(own)

        @pl.when(own)
        def _(j=j, c=c):
            pltpu.async_copy(out_hbm.at[pl.ds(c * CH, CH), :], feat.at[j],
                             sem_f.at[j])

            def idx_body(k, _):
                zv = zc[pl.ds(j * CH + k * 16, 16)]
                bv = bc[pl.ds(j * CH + k * 16, 16)]
                loc = (bv - g_lo) * NUM_ELEMENTS + zv - 1
                ok = (loc >= 0) & (loc < HALF)
                idx2[j, pl.ds(k * 16, 16)] = jnp.where(ok, loc, DUMMY)
                return 0

            lax.fori_loop(0, CH // 16, idx_body, 0)

    for i in range(ACC_PER_TILE // ZROWS):
        pltpu.make_async_copy(
            zerobuf, acc.at[pl.ds(sid * ACC_PER_TILE + i * ZROWS, ZROWS), :],
            sem_z0).wait()
    plsc.subcore_barrier()

    def scatter_desc(j, s):
        return pltpu.make_async_copy(feat.at[s], acc.at[idx2.at[j]],
                                     sem_sc.at[s])

    for j in range(CPT):
        c = j * NTILES + sid
        s = j % RING

        @pl.when(owned[j])
        def _(j=j, c=c, s=s):
            pltpu.make_async_copy(out_hbm.at[pl.ds(c * CH, CH), :], feat.at[s],
                                  sem_f.at[s]).wait()
            pltpu.async_copy(feat.at[s], acc.at[idx2.at[j]], sem_sc.at[s],
                             add=True)

        if j + RING < CPT:
            c2 = (j + RING) * NTILES + sid

            @pl.when(owned[j + RING])
            def _(j=j, c2=c2, s=s):
                @pl.when(owned[j])
                def _():
                    scatter_desc(j, s).wait()

                pltpu.async_copy(out_hbm.at[pl.ds(c2 * CH, CH), :], feat.at[s],
                                 sem_f.at[s])

    for j in range(CPT):
        s = j % RING
        if j + RING < CPT:
            not_refilled = jnp.logical_not(owned[j + RING])
            pred = jnp.logical_and(owned[j], not_refilled)
        else:
            pred = owned[j]

        @pl.when(pred)
        def _(j=j, s=s):
            scatter_desc(j, s).wait()

    plsc.subcore_barrier()

    pltpu.sync_copy(
        acc.at[pl.ds(sid * ACC_PER_TILE, ACC_PER_TILE), :],
        o_hbm.at[pl.ds(cid * HALF + sid * ACC_PER_TILE, ACC_PER_TILE), :])


@jax.jit
def _pool(out, z, batch):
    mesh = plsc.VectorSubcoreMesh(core_axis_name="c", subcore_axis_name="s",
                                  num_cores=2)
    f = functools.partial(
        pl.kernel,
        out_type=jax.ShapeDtypeStruct((SEGS, D), jnp.float32),
        mesh=mesh,
        scratch_types=[
            pltpu.VMEM((ZROWS, D), jnp.float32),
            pltpu.VMEM((CPT * CH,), jnp.int32),
            pltpu.VMEM((CPT * CH,), jnp.int32),
            pltpu.VMEM((CPT, CH), jnp.int32),
            pltpu.VMEM((RING, CH, D), jnp.float32),
            pltpu.VMEM_SHARED((HALF + 1, D), jnp.float32),
            pltpu.SemaphoreType.DMA,
            pltpu.SemaphoreType.DMA,
            pltpu.SemaphoreType.DMA((RING,)),
            pltpu.SemaphoreType.DMA((RING,)),
        ],
    )(_pool_kernel)
    return f(out, z, batch)


def kernel(out, z, batch):
    pooled = _pool(out, z.astype(jnp.int32), batch.astype(jnp.int32))
    return pooled.reshape(NUM_GRAPHS, NUM_ELEMENTS * D)
